# baseline (speedup 1.0000x reference)
import jax
import jax.numpy as jnp
from jax.experimental import pallas as pl

_B, _N, _D = 8, 2048, 3
_K = 16
_NSTRIP = 8
_S = _N // _NSTRIP


def _chamfer_body(aa_ref, bb_ref, out_ref):
    b = pl.program_id(0)
    aa = aa_ref[0]
    bb = bb_ref[0]

    srow = None
    cmin = None
    for r in range(_NSTRIP):
        a_strip = aa[r * _S:(r + 1) * _S, :]
        d = jax.lax.dot_general(
            a_strip, bb,
            dimension_numbers=(((1,), (0,)), ((), ())),
            preferred_element_type=jnp.float32,
        )
        rmin = jnp.min(d, axis=1, keepdims=True)
        cpart = jnp.min(d, axis=0, keepdims=True)
        sr = jnp.sum(rmin, axis=(0, 1), keepdims=True)
        srow = sr if srow is None else srow + sr
        cmin = cpart if cmin is None else jnp.minimum(cmin, cpart)

    s = srow + jnp.sum(cmin, axis=(0, 1), keepdims=True)

    @pl.when(b == 0)
    def _():
        out_ref[:, :] = jnp.zeros_like(s)

    out_ref[:, :] += s


def _hi_lo(x):
    hi = x.astype(jnp.bfloat16)
    lo = (x - hi.astype(jnp.float32)).astype(jnp.bfloat16)
    return hi, lo


def kernel(predict_pc, gt_pc):
    a = predict_pc
    bt = jnp.transpose(gt_pc, (0, 2, 1))

    a2hi, a2lo = _hi_lo(a * (-2.0))
    bhi, blo = _hi_lo(bt)
    nahi, nalo = _hi_lo(jnp.sum(a * a, axis=2))
    nbhi, nblo = _hi_lo(jnp.sum(bt * bt, axis=1))
    ones_a = jnp.ones((_B, _N), jnp.bfloat16)
    ones_b = jnp.ones((_B, _N), jnp.bfloat16)

    acols = []
    brows = []
    for k in range(_D):
        acols += [a2hi[..., k], a2hi[..., k], a2lo[..., k], a2lo[..., k]]
        brows += [bhi[:, k, :], blo[:, k, :], bhi[:, k, :], blo[:, k, :]]
    acols += [ones_a, ones_a, nahi, nalo]
    brows += [nbhi, nblo, ones_b, ones_b]
    aa = jnp.stack(acols, axis=2)
    bb = jnp.stack(brows, axis=1)

    out = pl.pallas_call(
        _chamfer_body,
        grid=(_B,),
        in_specs=[
            pl.BlockSpec((1, _N, _K), lambda b: (b, 0, 0)),
            pl.BlockSpec((1, _K, _N), lambda b: (b, 0, 0)),
        ],
        out_specs=pl.BlockSpec((1, 1), lambda b: (0, 0)),
        out_shape=jax.ShapeDtypeStruct((1, 1), jnp.float32),
    )(aa, bb)
    return out[0, 0] / (2.0 * _B * _N)


# speedup vs baseline: 4.7224x; 4.7224x over previous
import jax
import jax.numpy as jnp
from jax.experimental import pallas as pl

_B, _N, _D = 8, 2048, 3
_K = 16
_NSTRIP = 8
_S = _N // _NSTRIP


def _hi_lo(x):
    hi = x.astype(jnp.bfloat16)
    lo = (x - hi.astype(jnp.float32)).astype(jnp.bfloat16)
    return hi, lo


def _chamfer_body(a_ref, g_ref, out_ref):
    b = pl.program_id(0)
    at = a_ref[0]
    gt = g_ref[0]

    a2h, a2l = _hi_lo(at * (-2.0))
    bh, bl = _hi_lo(gt)
    nah, nal = _hi_lo(jnp.sum(at * at, axis=0, keepdims=True))
    nbh, nbl = _hi_lo(jnp.sum(gt * gt, axis=0, keepdims=True))
    one = jnp.ones((1, _N), jnp.bfloat16)

    aa = jnp.concatenate([a2h, a2l, a2h, a2l, one, one, nah, nal], axis=0)
    bb = jnp.concatenate([bh, bh, bl, bl, nbh, nbl, one, one], axis=0)

    srow = None
    cmin = None
    for r in range(_NSTRIP):
        a_strip = aa[:, r * _S:(r + 1) * _S]
        d = jax.lax.dot_general(
            a_strip, bb,
            dimension_numbers=(((0,), (0,)), ((), ())),
            preferred_element_type=jnp.float32,
        )
        rmin = jnp.min(d, axis=1, keepdims=True)
        cpart = jnp.min(d, axis=0, keepdims=True)
        sr = jnp.sum(rmin, axis=(0, 1), keepdims=True)
        srow = sr if srow is None else srow + sr
        cmin = cpart if cmin is None else jnp.minimum(cmin, cpart)

    s = srow + jnp.sum(cmin, axis=(0, 1), keepdims=True)

    @pl.when(b == 0)
    def _():
        out_ref[:, :] = jnp.zeros_like(s)

    out_ref[:, :] += s


def kernel(predict_pc, gt_pc):
    at = jnp.transpose(predict_pc, (0, 2, 1))
    gt = jnp.transpose(gt_pc, (0, 2, 1))
    out = pl.pallas_call(
        _chamfer_body,
        grid=(_B,),
        in_specs=[
            pl.BlockSpec((1, _D, _N), lambda b: (b, 0, 0)),
            pl.BlockSpec((1, _D, _N), lambda b: (b, 0, 0)),
        ],
        out_specs=pl.BlockSpec((1, 1), lambda b: (0, 0)),
        out_shape=jax.ShapeDtypeStruct((1, 1), jnp.float32),
    )(at, gt)
    return out[0, 0] / (2.0 * _B * _N)


# speedup vs baseline: 5.2242x; 1.1063x over previous
import jax
import jax.numpy as jnp
from jax.experimental import pallas as pl

_B, _N, _D = 8, 2048, 3
_K = 16
_NSTRIP = 8
_S = _N // _NSTRIP


def _hi_lo(x):
    hi = x.astype(jnp.bfloat16)
    lo = (x - hi.astype(jnp.float32)).astype(jnp.bfloat16)
    return hi, lo


def _chamfer_body(a_ref, g_ref, out_ref):
    s = None
    for b in range(_B):
        at = a_ref[b]
        gt = g_ref[b]

        a2h, a2l = _hi_lo(at * (-2.0))
        bh, bl = _hi_lo(gt)
        nah, nal = _hi_lo(jnp.sum(at * at, axis=0, keepdims=True))
        nbh, nbl = _hi_lo(jnp.sum(gt * gt, axis=0, keepdims=True))
        one = jnp.ones((1, _N), jnp.bfloat16)

        aa = jnp.concatenate([a2h, a2l, a2h, a2l, one, one, nah, nal],
                             axis=0)
        bb = jnp.concatenate([bh, bh, bl, bl, nbh, nbl, one, one], axis=0)

        srow = None
        cmin = None
        for r in range(_NSTRIP):
            a_strip = aa[:, r * _S:(r + 1) * _S]
            d = jax.lax.dot_general(
                a_strip, bb,
                dimension_numbers=(((0,), (0,)), ((), ())),
                preferred_element_type=jnp.float32,
            )
            rmin = jnp.min(d, axis=1, keepdims=True)
            cpart = jnp.min(d, axis=0, keepdims=True)
            sr = jnp.sum(rmin, axis=(0, 1), keepdims=True)
            srow = sr if srow is None else srow + sr
            cmin = cpart if cmin is None else jnp.minimum(cmin, cpart)

        sb = srow + jnp.sum(cmin, axis=(0, 1), keepdims=True)
        s = sb if s is None else s + sb

    out_ref[:, :] = s


def kernel(predict_pc, gt_pc):
    at = jnp.transpose(predict_pc, (0, 2, 1))
    gt = jnp.transpose(gt_pc, (0, 2, 1))
    out = pl.pallas_call(
        _chamfer_body,
        in_specs=[
            pl.BlockSpec((_B, _D, _N), lambda: (0, 0, 0)),
            pl.BlockSpec((_B, _D, _N), lambda: (0, 0, 0)),
        ],
        out_specs=pl.BlockSpec((1, 1), lambda: (0, 0)),
        out_shape=jax.ShapeDtypeStruct((1, 1), jnp.float32),
    )(at, gt)
    return out[0, 0] / (2.0 * _B * _N)
